# baseline (speedup 1.0000x reference)
import jax
import jax.numpy as jnp
from jax import lax
from jax.experimental import pallas as pl
from jax.experimental.pallas import tpu as pltpu
from jax.experimental.pallas import tpu_sc as plsc

NMS_THR = 0.5
EPS = 1e-06
_N = 5000
_G = 100
_L = 16
_NP = 5120
_NT = 16
_CPT = _NP // _L // _NT
_NEG = -1e30
_LN2 = 0.6931471805599453


def _log16(x):
    b = lax.bitcast_convert_type(x, jnp.int32)
    e = lax.shift_right_logical(b, 23) - 127
    mb = lax.bitwise_or(lax.bitwise_and(b, 0x007FFFFF), 0x3F800000)
    m = lax.bitcast_convert_type(mb, jnp.float32)
    z = (m - 1.0) / (m + 1.0)
    zz = z * z
    lm = z * (2.0 + zz * (2.0 / 3.0 + zz * (2.0 / 5.0 + zz * (2.0 / 7.0))))
    return e.astype(jnp.float32) * _LN2 + lm


def _sc_body(px1, py1, px2, py2, ps, pg, hx1, hy1, hx2, hy2, out,
             vx1, vy1, vx2, vy2, vs, vg, valive, varea,
             vb1, vb2, vb3, vb4, vgt1, vgt2, vgt3, vgt4, vrec,
             vpart, vbuf, vout, shared, sem):
    i32 = jnp.int32
    f32 = jnp.float32
    ci = lax.axis_index("c")
    si = lax.axis_index("s")

    @pl.when(ci == 0)
    def _():
        pltpu.sync_copy(px1, vx1)
        pltpu.sync_copy(py1, vy1)
        pltpu.sync_copy(px2, vx2)
        pltpu.sync_copy(py2, vy2)
        pltpu.sync_copy(ps, vs)
        pltpu.sync_copy(pg, vg)
        pltpu.sync_copy(hx1, vgt1)
        pltpu.sync_copy(hy1, vgt2)
        pltpu.sync_copy(hx2, vgt3)
        pltpu.sync_copy(hy2, vgt4)

        lane = lax.iota(i32, _L)
        base = si * _CPT

        def f16(x):
            return jnp.full((_L,), x, f32)

        def i16(x):
            return jnp.full((_L,), x, i32)

        def init_rec(k, _):
            vrec[pl.ds(k * _L, _L)] = i16(-1)
            return 0

        lax.fori_loop(0, 128 // _L, init_rec, 0)

        def pre(jj, carry):
            mval, midx = carry
            j = base + jj
            sl = pl.ds(j * _L, _L)
            flat = j * _L + lane
            x1c = vx1[sl]
            y1c = vy1[sl]
            x2c = vx2[sl]
            y2c = vy2[sl]
            sc = vs[sl]
            gc = vg[sl]
            varea[sl] = (x2c - x1c + 1.0) * (y2c - y1c + 1.0)
            gidx = jnp.maximum(gc, 0)
            vb1[sl] = plsc.load_gather(vgt1, [gidx])
            vb2[sl] = plsc.load_gather(vgt2, [gidx])
            vb3[sl] = plsc.load_gather(vgt3, [gidx])
            vb4[sl] = plsc.load_gather(vgt4, [gidx])
            al = gc >= 0
            valive[sl] = al.astype(i32)
            msd = jnp.where(al, sc, _NEG)
            upd = msd >= mval
            return (jnp.where(upd, msd, mval), jnp.where(upd, flat, midx))

        mval, midx = lax.fori_loop(0, _CPT, pre, (f16(_NEG), i16(-1)))

        def publish(p, mv, midx_, remf, cn2f, pushf):
            vec = jnp.where(lane == 0, f16(mv),
                  jnp.where(lane == 1, f16(midx_.astype(f32)),
                  jnp.where(lane == 2, f16(remf),
                  jnp.where(lane == 3, f16(cn2f), f16(pushf)))))
            vpart[...] = vec
            return pltpu.async_copy(
                vpart, shared.at[pl.ds(p * _NT * _L + si * _L, _L)], sem)

        def reduce_partials(p):
            plsc.subcore_barrier()
            pltpu.sync_copy(shared.at[pl.ds(p * _NT * _L, _NT * _L)], vbuf)
            g0 = lane * _L
            mvv = plsc.load_gather(vbuf, [g0])
            idv = plsc.load_gather(vbuf, [g0 + 1]).astype(i32)
            rmv = plsc.load_gather(vbuf, [g0 + 2])
            cnv = plsc.load_gather(vbuf, [g0 + 3])
            psv = plsc.load_gather(vbuf, [g0 + 4])
            m = jnp.max(mvv)
            iw = jnp.maximum(jnp.max(jnp.where(mvv == m, idv, -1)), 0)
            return m, iw, jnp.sum(rmv), jnp.sum(cnv), jnp.sum(psv)

        tmv = jnp.max(mval)
        tix = jnp.max(jnp.where(mval == tmv, midx, -1))
        publish(0, tmv, tix, 0.0, 0.0, 0.0).wait()
        m0, i0, _, _, _ = reduce_partials(0)
        found0 = m0 > 0.5 * _NEG

        def winner(i):
            iv = i16(i)
            return (plsc.load_gather(vx1, [iv]), plsc.load_gather(vy1, [iv]),
                    plsc.load_gather(vx2, [iv]), plsc.load_gather(vy2, [iv]),
                    plsc.load_gather(vs, [iv]), plsc.load_gather(vg, [iv]))

        w0 = winner(i0)

        def cond(st):
            return st[0]

        def body(st):
            (_, p, i, wx1, wy1, wx2, wy2, ws, wg,
             tot_pull, tot_push, pull_cnt, push_cnt) = st
            area_i = (wx2 - wx1 + 1.0) * (wy2 - wy1 + 1.0)

            ga1 = plsc.load_gather(vgt1, [wg])
            ga2 = plsc.load_gather(vgt2, [wg])
            ga3 = plsc.load_gather(vgt3, [wg])
            ga4 = plsc.load_gather(vgt4, [wg])
            ga_area = (ga3 - ga1 + 1.0) * (ga4 - ga2 + 1.0)

            def chunk(jj, c):
                j = base + jj
                sl = pl.ds(j * _L, _L)
                flat = j * _L + lane
                alc = valive[sl] > 0

                def live(c):
                    rem, cn2, pacc, nmv, nmi = c
                    x1c = vx1[sl]
                    y1c = vy1[sl]
                    x2c = vx2[sl]
                    y2c = vy2[sl]
                    sc = vs[sl]
                    arc = varea[sl]
                    alive2 = alc & (flat != i)
                    rem = rem + jnp.where(alive2, 1.0, 0.0)
                    w = jnp.maximum(jnp.minimum(wx2, x2c) - jnp.maximum(wx1, x1c) + 1.0, 0.0)
                    h = jnp.maximum(jnp.minimum(wy2, y2c) - jnp.maximum(wy1, y1c) + 1.0, 0.0)
                    ovl = w * h
                    row = ovl / (area_i + arc - ovl)
                    ovm = alive2 & (row > NMS_THR)
                    alnew = alive2 & (row <= NMS_THR)
                    valive[sl] = alnew.astype(i32)
                    msd = jnp.where(alnew, sc, _NEG)
                    upd = msd >= nmv
                    nmv = jnp.where(upd, msd, nmv)
                    nmi = jnp.where(upd, flat, nmi)

                    def hit(c2):
                        cn2, pacc = c2
                        gc = vg[sl]
                        gb1 = vb1[sl]
                        gb2 = vb2[sl]
                        gb3 = vb3[sl]
                        gb4 = vb4[sl]
                        garea_c = (gb3 - gb1 + 1.0) * (gb4 - gb2 + 1.0)
                        gw = jnp.maximum(jnp.minimum(ga3, gb3) - jnp.maximum(ga1, gb1) + 1.0, 0.0)
                        gh = jnp.maximum(jnp.minimum(ga4, gb4) - jnp.maximum(ga2, gb2) + 1.0, 0.0)
                        govl = gw * gh
                        giou = govl / (ga_area + garea_c - govl)
                        pm2 = ovm & (gc != wg) & (row > giou)
                        cn2 = cn2 + jnp.where(pm2, 1.0, 0.0)
                        plv = -_log16(1.0 + NMS_THR - row) * sc
                        pacc = pacc + jnp.where(pm2, plv, 0.0)
                        return (cn2, pacc)

                    cn2, pacc = lax.cond(jnp.any(ovm), hit, lambda c2: c2,
                                         (cn2, pacc))
                    return (rem, cn2, pacc, nmv, nmi)

                return lax.cond(jnp.any(alc), live, lambda c_: c_, c)

            rem16, cn216, pacc16, nmv, nmi = lax.fori_loop(
                0, _CPT, chunk,
                (f16(0.0), f16(0.0), f16(0.0), f16(_NEG), i16(-1)))

            tmv2 = jnp.max(nmv)
            tix2 = jnp.max(jnp.where(nmv == tmv2, nmi, -1))
            desc = publish(p, tmv2, tix2, jnp.sum(rem16), jnp.sum(cn216),
                           jnp.sum(pacc16))

            rep = plsc.load_gather(vrec, [wg])
            has = rep >= 0
            rr = jnp.maximum(rep, 0)
            bx1 = plsc.load_gather(vx1, [rr])
            by1 = plsc.load_gather(vy1, [rr])
            bx2 = plsc.load_gather(vx2, [rr])
            by2 = plsc.load_gather(vy2, [rr])
            rarea = (bx2 - bx1 + 1.0) * (by2 - by1 + 1.0)
            wv = jnp.maximum(jnp.minimum(wx2, bx2) - jnp.maximum(wx1, bx1) + 1.0, 0.0)
            hv = jnp.maximum(jnp.minimum(wy2, by2) - jnp.maximum(wy1, by1) + 1.0, 0.0)
            ovl0 = wv * hv
            iou_ir = ovl0 / (area_i + rarea - ovl0)
            ms = jnp.maximum(iou_ir, EPS)
            lp = _log16(ms)
            pull = jnp.where(has, -lp * ws, 0.0)
            plsc.store_scatter(vrec, [wg], i16(i), mask=(lane == 0) & (rep < 0))

            desc.wait()
            m, inext, remaining, cnt2, push_sum = reduce_partials(p)

            push = jnp.where(f16(cnt2) > 0, f16(push_sum) / f16(cnt2), 0.0)
            cont = remaining > 0
            tot_pull = tot_pull + jnp.where(cont, pull, 0.0)
            tot_push = tot_push + jnp.where(cont, push, 0.0)
            pull_cnt = pull_cnt + jnp.where(has, 1.0, 0.0)
            push_cnt = push_cnt + jnp.where(cont, f16(cnt2), f16(0.0))

            found = m > 0.5 * _NEG
            wn = winner(inext)
            return (found, 1 - p, inext) + wn + (tot_pull, tot_push,
                                                 pull_cnt, push_cnt)

        init = (found0, jnp.int32(1), i0) + w0 + (f16(0.0), f16(0.0),
                                                  f16(0.0), f16(0.0))
        st = lax.while_loop(cond, body, init)
        tot_pull, tot_push, pull_cnt, push_cnt = st[9], st[10], st[11], st[12]
        push_loss = tot_push / (push_cnt + EPS)
        pull_loss = tot_pull / (pull_cnt + EPS)

        @pl.when(si == 0)
        def _():
            vout[...] = jnp.where(lane == 0, push_loss,
                                  jnp.where(lane == 1, pull_loss, f16(0.0)))
            pltpu.sync_copy(vout, out)


@jax.jit
def _run_sc(g0, gt, props):
    f32 = jnp.float32
    i32 = jnp.int32
    pad = _NP - _N
    p = jnp.pad(props, ((0, pad), (0, 0)))
    g = jnp.pad(g0.astype(i32), (0, pad), constant_values=-1)
    gtp = jnp.pad(gt, ((0, 128 - _G), (0, 0)))

    mesh = plsc.VectorSubcoreMesh(core_axis_name="c", subcore_axis_name="s",
                                  num_cores=2, num_subcores=16)
    fn = pl.kernel(
        _sc_body,
        out_type=jax.ShapeDtypeStruct((_L,), f32),
        mesh=mesh,
        compiler_params=pltpu.CompilerParams(needs_layout_passes=False),
        scratch_types=[
            pltpu.VMEM((_NP,), f32), pltpu.VMEM((_NP,), f32),
            pltpu.VMEM((_NP,), f32), pltpu.VMEM((_NP,), f32),
            pltpu.VMEM((_NP,), f32), pltpu.VMEM((_NP,), i32),
            pltpu.VMEM((_NP,), i32), pltpu.VMEM((_NP,), f32),
            pltpu.VMEM((_NP,), f32), pltpu.VMEM((_NP,), f32),
            pltpu.VMEM((_NP,), f32), pltpu.VMEM((_NP,), f32),
            pltpu.VMEM((128,), f32), pltpu.VMEM((128,), f32),
            pltpu.VMEM((128,), f32), pltpu.VMEM((128,), f32),
            pltpu.VMEM((128,), i32),
            pltpu.VMEM((_L,), f32), pltpu.VMEM((_NT * _L,), f32),
            pltpu.VMEM((_L,), f32),
            pltpu.VMEM_SHARED((2 * _NT * _L,), f32),
            pltpu.SemaphoreType.DMA,
        ],
    )
    out = fn(p[:, 0], p[:, 1], p[:, 2], p[:, 3], p[:, 4], g,
             gtp[:, 0], gtp[:, 1], gtp[:, 2], gtp[:, 3])
    return out[0], out[1]


def kernel(gt_inds, anchor_gt_inds, gt_bboxes, proposal_list):
    g0 = anchor_gt_inds[0]
    gt = gt_bboxes[0].astype(jnp.float32)
    props = proposal_list[0].astype(jnp.float32)
    push, pull = _run_sc(g0, gt, props)
    return (push, pull)


# speedup vs baseline: 2.6943x; 2.6943x over previous
import jax
import jax.numpy as jnp
from jax import lax
from jax.experimental import pallas as pl
from jax.experimental.pallas import tpu as pltpu
from jax.experimental.pallas import tpu_sc as plsc

NMS_THR = 0.5
EPS = 1e-06
_N = 5000
_G = 100
_L = 16
_NP = 5120
_NT = 16
_CPT = _NP // _L // _NT
_NEG = -1e30
_LN2 = 0.6931471805599453


def _log16(x):
    b = lax.bitcast_convert_type(x, jnp.int32)
    e = lax.shift_right_logical(b, 23) - 127
    mb = lax.bitwise_or(lax.bitwise_and(b, 0x007FFFFF), 0x3F800000)
    m = lax.bitcast_convert_type(mb, jnp.float32)
    z = (m - 1.0) / (m + 1.0)
    zz = z * z
    lm = z * (2.0 + zz * (2.0 / 3.0 + zz * (2.0 / 5.0 + zz * (2.0 / 7.0))))
    return e.astype(jnp.float32) * _LN2 + lm


def _sc_body(px1, py1, px2, py2, ps, pg, hx1, hy1, hx2, hy2, out,
             vx1, vy1, vx2, vy2, vs, vg, valive, varea,
             vb1, vb2, vb3, vb4, vgt1, vgt2, vgt3, vgt4, vrec,
             vpart, vbuf, vout, shared, sem):
    i32 = jnp.int32
    f32 = jnp.float32
    ci = lax.axis_index("c")
    si = lax.axis_index("s")

    @pl.when(ci == 0)
    def _():
        pltpu.sync_copy(px1, vx1)
        pltpu.sync_copy(py1, vy1)
        pltpu.sync_copy(px2, vx2)
        pltpu.sync_copy(py2, vy2)
        pltpu.sync_copy(ps, vs)
        pltpu.sync_copy(pg, vg)
        pltpu.sync_copy(hx1, vgt1)
        pltpu.sync_copy(hy1, vgt2)
        pltpu.sync_copy(hx2, vgt3)
        pltpu.sync_copy(hy2, vgt4)

        lane = lax.iota(i32, _L)
        base = si * _CPT

        def f16(x):
            return jnp.full((_L,), x, f32)

        def i16(x):
            return jnp.full((_L,), x, i32)

        def init_rec(k, _):
            vrec[pl.ds(k * _L, _L)] = i16(-1)
            return 0

        lax.fori_loop(0, 128 // _L, init_rec, 0)

        def pre(jj, carry):
            mval, midx = carry
            j = base + jj
            sl = pl.ds(j * _L, _L)
            flat = j * _L + lane
            x1c = vx1[sl]
            y1c = vy1[sl]
            x2c = vx2[sl]
            y2c = vy2[sl]
            sc = vs[sl]
            gc = vg[sl]
            varea[sl] = (x2c - x1c + 1.0) * (y2c - y1c + 1.0)
            gidx = jnp.maximum(gc, 0)
            vb1[sl] = plsc.load_gather(vgt1, [gidx])
            vb2[sl] = plsc.load_gather(vgt2, [gidx])
            vb3[sl] = plsc.load_gather(vgt3, [gidx])
            vb4[sl] = plsc.load_gather(vgt4, [gidx])
            al = gc >= 0
            valive[sl] = al.astype(i32)
            msd = jnp.where(al, sc, _NEG)
            upd = msd >= mval
            return (jnp.where(upd, msd, mval), jnp.where(upd, flat, midx))

        mval, midx = plsc.parallel_loop(
            0, _CPT, 1, unroll=2, carry=(f16(_NEG), i16(-1)))(pre)

        def publish(p, mv, midx_, remf, cn2f, pushf):
            vec = jnp.where(lane == 0, f16(mv),
                  jnp.where(lane == 1, f16(midx_.astype(f32)),
                  jnp.where(lane == 2, f16(remf),
                  jnp.where(lane == 3, f16(cn2f), f16(pushf)))))
            vpart[...] = vec
            return pltpu.async_copy(
                vpart, shared.at[pl.ds(p * _NT * _L + si * _L, _L)], sem)

        def reduce_partials(p):
            plsc.subcore_barrier()
            pltpu.sync_copy(shared.at[pl.ds(p * _NT * _L, _NT * _L)], vbuf)
            g0 = lane * _L
            mvv = plsc.load_gather(vbuf, [g0])
            idv = plsc.load_gather(vbuf, [g0 + 1]).astype(i32)
            rmv = plsc.load_gather(vbuf, [g0 + 2])
            cnv = plsc.load_gather(vbuf, [g0 + 3])
            psv = plsc.load_gather(vbuf, [g0 + 4])
            m = jnp.max(mvv)
            iw = jnp.maximum(jnp.max(jnp.where(mvv == m, idv, -1)), 0)
            return m, iw, jnp.sum(rmv), jnp.sum(cnv), jnp.sum(psv)

        tmv = jnp.max(mval)
        tix = jnp.max(jnp.where(mval == tmv, midx, -1))
        publish(0, tmv, tix, 0.0, 0.0, 0.0).wait()
        m0, i0, _, _, _ = reduce_partials(0)
        found0 = m0 > 0.5 * _NEG

        def winner(i):
            iv = i16(i)
            return (plsc.load_gather(vx1, [iv]), plsc.load_gather(vy1, [iv]),
                    plsc.load_gather(vx2, [iv]), plsc.load_gather(vy2, [iv]),
                    plsc.load_gather(vs, [iv]), plsc.load_gather(vg, [iv]))

        w0 = winner(i0)

        def cond(st):
            return st[0]

        def body(st):
            (_, p, i, wx1, wy1, wx2, wy2, ws, wg,
             tot_pull, tot_push, pull_cnt, push_cnt) = st
            area_i = (wx2 - wx1 + 1.0) * (wy2 - wy1 + 1.0)

            ga1 = plsc.load_gather(vgt1, [wg])
            ga2 = plsc.load_gather(vgt2, [wg])
            ga3 = plsc.load_gather(vgt3, [wg])
            ga4 = plsc.load_gather(vgt4, [wg])
            ga_area = (ga3 - ga1 + 1.0) * (ga4 - ga2 + 1.0)

            def chunk(jj, c):
                rem, cn2, pacc, nmv, nmi = c
                j = base + jj
                sl = pl.ds(j * _L, _L)
                flat = j * _L + lane
                x1c = vx1[sl]
                y1c = vy1[sl]
                x2c = vx2[sl]
                y2c = vy2[sl]
                sc = vs[sl]
                gc = vg[sl]
                arc = varea[sl]
                alc = valive[sl] > 0
                alive2 = alc & (flat != i)
                rem = rem + jnp.where(alive2, 1.0, 0.0)
                w = jnp.maximum(jnp.minimum(wx2, x2c) - jnp.maximum(wx1, x1c) + 1.0, 0.0)
                h = jnp.maximum(jnp.minimum(wy2, y2c) - jnp.maximum(wy1, y1c) + 1.0, 0.0)
                ovl = w * h
                row = ovl / (area_i + arc - ovl)
                gb1 = vb1[sl]
                gb2 = vb2[sl]
                gb3 = vb3[sl]
                gb4 = vb4[sl]
                garea_c = (gb3 - gb1 + 1.0) * (gb4 - gb2 + 1.0)
                gw = jnp.maximum(jnp.minimum(ga3, gb3) - jnp.maximum(ga1, gb1) + 1.0, 0.0)
                gh = jnp.maximum(jnp.minimum(ga4, gb4) - jnp.maximum(ga2, gb2) + 1.0, 0.0)
                govl = gw * gh
                giou = govl / (ga_area + garea_c - govl)
                ovm = alive2 & (row > NMS_THR)
                pm2 = ovm & (gc != wg) & (row > giou)
                cn2 = cn2 + jnp.where(pm2, 1.0, 0.0)
                plv = -_log16(1.0 + NMS_THR - row) * sc
                pacc = pacc + jnp.where(pm2, plv, 0.0)
                alnew = alive2 & (row <= NMS_THR)
                valive[sl] = alnew.astype(i32)
                msd = jnp.where(alnew, sc, _NEG)
                upd = msd >= nmv
                return (rem, cn2, pacc,
                        jnp.where(upd, msd, nmv), jnp.where(upd, flat, nmi))

            rem16, cn216, pacc16, nmv, nmi = plsc.parallel_loop(
                0, _CPT, 1, unroll=2,
                carry=(f16(0.0), f16(0.0), f16(0.0), f16(_NEG), i16(-1)),
            )(chunk)

            tmv2 = jnp.max(nmv)
            tix2 = jnp.max(jnp.where(nmv == tmv2, nmi, -1))
            desc = publish(p, tmv2, tix2, jnp.sum(rem16), jnp.sum(cn216),
                           jnp.sum(pacc16))

            rep = plsc.load_gather(vrec, [wg])
            has = rep >= 0
            rr = jnp.maximum(rep, 0)
            bx1 = plsc.load_gather(vx1, [rr])
            by1 = plsc.load_gather(vy1, [rr])
            bx2 = plsc.load_gather(vx2, [rr])
            by2 = plsc.load_gather(vy2, [rr])
            rarea = (bx2 - bx1 + 1.0) * (by2 - by1 + 1.0)
            wv = jnp.maximum(jnp.minimum(wx2, bx2) - jnp.maximum(wx1, bx1) + 1.0, 0.0)
            hv = jnp.maximum(jnp.minimum(wy2, by2) - jnp.maximum(wy1, by1) + 1.0, 0.0)
            ovl0 = wv * hv
            iou_ir = ovl0 / (area_i + rarea - ovl0)
            ms = jnp.maximum(iou_ir, EPS)
            lp = _log16(ms)
            pull = jnp.where(has, -lp * ws, 0.0)
            plsc.store_scatter(vrec, [wg], i16(i), mask=(lane == 0) & (rep < 0))

            desc.wait()
            m, inext, remaining, cnt2, push_sum = reduce_partials(p)

            push = jnp.where(f16(cnt2) > 0, f16(push_sum) / f16(cnt2), 0.0)
            cont = remaining > 0
            tot_pull = tot_pull + jnp.where(cont, pull, 0.0)
            tot_push = tot_push + jnp.where(cont, push, 0.0)
            pull_cnt = pull_cnt + jnp.where(has, 1.0, 0.0)
            push_cnt = push_cnt + jnp.where(cont, f16(cnt2), f16(0.0))

            found = m > 0.5 * _NEG
            wn = winner(inext)
            return (found, 1 - p, inext) + wn + (tot_pull, tot_push,
                                                 pull_cnt, push_cnt)

        init = (found0, jnp.int32(1), i0) + w0 + (f16(0.0), f16(0.0),
                                                  f16(0.0), f16(0.0))
        st = lax.while_loop(cond, body, init)
        tot_pull, tot_push, pull_cnt, push_cnt = st[9], st[10], st[11], st[12]
        push_loss = tot_push / (push_cnt + EPS)
        pull_loss = tot_pull / (pull_cnt + EPS)

        @pl.when(si == 0)
        def _():
            vout[...] = jnp.where(lane == 0, push_loss,
                                  jnp.where(lane == 1, pull_loss, f16(0.0)))
            pltpu.sync_copy(vout, out)


@jax.jit
def _run_sc(g0, gt, props):
    f32 = jnp.float32
    i32 = jnp.int32
    pad = _NP - _N
    p = jnp.pad(props, ((0, pad), (0, 0)))
    g = jnp.pad(g0.astype(i32), (0, pad), constant_values=-1)
    gtp = jnp.pad(gt, ((0, 128 - _G), (0, 0)))

    mesh = plsc.VectorSubcoreMesh(core_axis_name="c", subcore_axis_name="s",
                                  num_cores=2, num_subcores=16)
    fn = pl.kernel(
        _sc_body,
        out_type=jax.ShapeDtypeStruct((_L,), f32),
        mesh=mesh,
        compiler_params=pltpu.CompilerParams(needs_layout_passes=False),
        scratch_types=[
            pltpu.VMEM((_NP,), f32), pltpu.VMEM((_NP,), f32),
            pltpu.VMEM((_NP,), f32), pltpu.VMEM((_NP,), f32),
            pltpu.VMEM((_NP,), f32), pltpu.VMEM((_NP,), i32),
            pltpu.VMEM((_NP,), i32), pltpu.VMEM((_NP,), f32),
            pltpu.VMEM((_NP,), f32), pltpu.VMEM((_NP,), f32),
            pltpu.VMEM((_NP,), f32), pltpu.VMEM((_NP,), f32),
            pltpu.VMEM((128,), f32), pltpu.VMEM((128,), f32),
            pltpu.VMEM((128,), f32), pltpu.VMEM((128,), f32),
            pltpu.VMEM((128,), i32),
            pltpu.VMEM((_L,), f32), pltpu.VMEM((_NT * _L,), f32),
            pltpu.VMEM((_L,), f32),
            pltpu.VMEM_SHARED((2 * _NT * _L,), f32),
            pltpu.SemaphoreType.DMA,
        ],
    )
    out = fn(p[:, 0], p[:, 1], p[:, 2], p[:, 3], p[:, 4], g,
             gtp[:, 0], gtp[:, 1], gtp[:, 2], gtp[:, 3])
    return out[0], out[1]


def kernel(gt_inds, anchor_gt_inds, gt_bboxes, proposal_list):
    g0 = anchor_gt_inds[0]
    gt = gt_bboxes[0].astype(jnp.float32)
    props = proposal_list[0].astype(jnp.float32)
    push, pull = _run_sc(g0, gt, props)
    return (push, pull)
